# baseline (speedup 1.0000x reference)
import functools

import jax
import jax.numpy as jnp
from jax import lax
from jax.experimental import pallas as pl
from jax.experimental.pallas import tpu as pltpu
from jax.experimental.pallas import tpu_sc as plsc

NUM_CLASSES = 12
NFEAT = 16
NPROP = NFEAT - 1
NOUT = NUM_CLASSES + NPROP
LANES = 16
NCOL = 128
NUM_WORKERS = 32


def _sc_body(xt_hbm, ot_hbm, xv, ov, sem_a, sem_b, sem_o, batch):
    cid = lax.axis_index("c")
    sid = lax.axis_index("s")
    wid = sid * 2 + cid
    n0 = wid * NCOL
    halff = NFEAT // 2
    nt = NCOL // LANES

    one = jnp.full((LANES,), 1.0, jnp.float32)
    zero = jnp.zeros((LANES,), jnp.float32)
    cls = [jnp.full((LANES,), float(c), jnp.float32) for c in range(NUM_CLASSES)]

    cin0 = pltpu.async_copy(
        xt_hbm.at[:, pl.ds(0, halff), pl.ds(n0, NCOL)],
        xv.at[:, pl.ds(0, halff)],
        sem_a,
    )
    cin1 = pltpu.async_copy(
        xt_hbm.at[:, pl.ds(halff, halff), pl.ds(n0, NCOL)],
        xv.at[:, pl.ds(halff, halff)],
        sem_b,
    )

    def onehot_planes(c0, c1):
        def body(i, _):
            b = i >> 3
            sl = pl.ds((i & 7) * LANES, LANES)
            ids = xv[b, 0, sl]
            for c in range(c0, c1):
                ov[c, b, sl] = jnp.where(ids == cls[c], one, zero)
            return ()

        plsc.parallel_loop(0, batch * nt, 1, unroll=2, carry=())(body)

    def prop_planes(j0, j1):
        def body(i, _):
            j = j0 + (i >> 7)
            b = (i >> 3) & (batch - 1)
            sl = pl.ds((i & 7) * LANES, LANES)
            ov[NUM_CLASSES + j, b, sl] = xv[b, 1 + j, sl]
            return ()

        plsc.parallel_loop(0, (j1 - j0) * batch * nt, 1, unroll=4, carry=())(
            body
        )

    def out_group(p0, p1):
        return pltpu.async_copy(
            ov.at[pl.ds(p0, p1 - p0)],
            ot_hbm.at[pl.ds(p0, p1 - p0), :, pl.ds(n0, NCOL)],
            sem_o,
        )

    cin0.wait()
    onehot_planes(0, 6)
    g0 = out_group(0, 6)
    onehot_planes(6, 12)
    g1 = out_group(6, 12)
    prop_planes(0, 7)
    g2 = out_group(12, 19)
    cin1.wait()
    prop_planes(7, 11)
    g3 = out_group(19, 23)
    prop_planes(11, 15)
    g4 = out_group(23, 27)
    for g in (g0, g1, g2, g3, g4):
        g.wait()


def kernel(X):
    B, N, F = X.shape
    assert F == NFEAT
    assert N % (NUM_WORKERS * NCOL) == 0 or (B * N) % (NUM_WORKERS * NCOL) == 0

    xt = jnp.transpose(X, (0, 2, 1))
    mesh = plsc.VectorSubcoreMesh(core_axis_name="c", subcore_axis_name="s")
    ot = pl.kernel(
        functools.partial(_sc_body, batch=B),
        out_type=jax.ShapeDtypeStruct((NOUT, B, N), jnp.float32),
        mesh=mesh,
        compiler_params=pltpu.CompilerParams(
            needs_layout_passes=False, use_tc_tiling_on_sc=True
        ),
        scratch_types=[
            pltpu.VMEM((B, NFEAT, NCOL), jnp.float32),
            pltpu.VMEM((NOUT, B, NCOL), jnp.float32),
            pltpu.SemaphoreType.DMA,
            pltpu.SemaphoreType.DMA,
            pltpu.SemaphoreType.DMA,
        ],
    )(xt)
    return jnp.transpose(ot, (1, 2, 0))


# speedup vs baseline: 1.0018x; 1.0018x over previous
import functools

import jax
import jax.numpy as jnp
from jax import lax
from jax.experimental import pallas as pl
from jax.experimental.pallas import tpu as pltpu
from jax.experimental.pallas import tpu_sc as plsc

NUM_CLASSES = 12
NFEAT = 16
NPROP = NFEAT - 1
NOUT = NUM_CLASSES + NPROP
LANES = 16
NCOL = 128
NUM_WORKERS = 32


def _sc_body(xt_hbm, ot_hbm, xv, ov, sem_a, sem_b, sem_o, batch):
    cid = lax.axis_index("c")
    sid = lax.axis_index("s")
    wid = sid * 2 + cid
    n0 = wid * NCOL
    halff = NFEAT // 2
    nt = NCOL // LANES

    one = jnp.full((LANES,), 1.0, jnp.float32)
    zero = jnp.zeros((LANES,), jnp.float32)
    cls = [jnp.full((LANES,), float(c), jnp.float32) for c in range(NUM_CLASSES)]

    cin0 = pltpu.async_copy(
        xt_hbm.at[:, pl.ds(0, halff), pl.ds(n0, NCOL)],
        xv.at[:, pl.ds(0, halff)],
        sem_a,
    )
    cin1 = pltpu.async_copy(
        xt_hbm.at[:, pl.ds(halff, halff), pl.ds(n0, NCOL)],
        xv.at[:, pl.ds(halff, halff)],
        sem_b,
    )

    def onehot_planes(c0, c1):
        def body(i, _):
            b = i >> 3
            sl = pl.ds((i & 7) * LANES, LANES)
            ids = xv[b, 0, sl]
            for c in range(c0, c1):
                ov[c, b, sl] = jnp.where(ids == cls[c], one, zero)
            return ()

        plsc.parallel_loop(0, batch * nt, 1, unroll=2, carry=())(body)

    def prop_planes(j0, j1):
        def body(i, _):
            j = j0 + (i >> 7)
            b = (i >> 3) & (batch - 1)
            sl = pl.ds((i & 7) * LANES, LANES)
            ov[NUM_CLASSES + j, b, sl] = xv[b, 1 + j, sl]
            return ()

        plsc.parallel_loop(0, (j1 - j0) * batch * nt, 1, unroll=4, carry=())(
            body
        )

    def out_group(p0, p1):
        return pltpu.async_copy(
            ov.at[pl.ds(p0, p1 - p0)],
            ot_hbm.at[pl.ds(p0, p1 - p0), :, pl.ds(n0, NCOL)],
            sem_o,
        )

    cin0.wait()
    onehot_planes(0, 6)
    g0 = out_group(0, 6)
    onehot_planes(6, 12)
    g1 = out_group(6, 12)
    prop_planes(0, 7)
    g2 = out_group(12, 19)
    cin1.wait()
    prop_planes(7, 11)
    g3 = out_group(19, 23)
    prop_planes(11, 15)
    g4 = out_group(23, 27)
    for g in (g0, g1, g2, g3, g4):
        g.wait()


def kernel(X):
    B, N, F = X.shape
    assert F == NFEAT
    assert N % (NUM_WORKERS * NCOL) == 0 or (B * N) % (NUM_WORKERS * NCOL) == 0

    xt = jnp.transpose(X, (0, 2, 1))
    mesh = plsc.VectorSubcoreMesh(core_axis_name="c", subcore_axis_name="s")
    ot = pl.kernel(
        functools.partial(_sc_body, batch=B),
        out_type=jax.ShapeDtypeStruct((NOUT, B, N), jnp.float32),
        mesh=mesh,
        compiler_params=pltpu.CompilerParams(
            needs_layout_passes=False,
            use_tc_tiling_on_sc=True,
            skip_device_barrier=True,
        ),
        scratch_types=[
            pltpu.VMEM((B, NFEAT, NCOL), jnp.float32),
            pltpu.VMEM((NOUT, B, NCOL), jnp.float32),
            pltpu.SemaphoreType.DMA,
            pltpu.SemaphoreType.DMA,
            pltpu.SemaphoreType.DMA,
        ],
    )(xt)
    return jnp.transpose(ot, (1, 2, 0))


# speedup vs baseline: 1.0358x; 1.0339x over previous
import functools

import jax
import jax.numpy as jnp
from jax import lax
from jax.experimental import pallas as pl
from jax.experimental.pallas import tpu as pltpu
from jax.experimental.pallas import tpu_sc as plsc

NUM_CLASSES = 12
NFEAT = 16
NPROP = NFEAT - 1
NOUT = NUM_CLASSES + NPROP
LANES = 16
NCOL = 128
NUM_WORKERS = 32


def _sc_body(xt_hbm, ot_hbm, xv, ov, sem_a, sem_b, sem_o, batch):
    cid = lax.axis_index("c")
    sid = lax.axis_index("s")
    wid = sid * 2 + cid
    n0 = wid * NCOL
    halff = NFEAT // 2
    nt = NCOL // LANES

    one = jnp.full((LANES,), 1.0, jnp.float32)
    zero = jnp.zeros((LANES,), jnp.float32)
    cls = [jnp.full((LANES,), float(c), jnp.float32) for c in range(NUM_CLASSES)]

    cin0 = pltpu.async_copy(
        xt_hbm.at[:, pl.ds(0, halff), pl.ds(n0, NCOL)],
        xv.at[:, pl.ds(0, halff)],
        sem_a,
    )
    cin1 = pltpu.async_copy(
        xt_hbm.at[:, pl.ds(halff, halff), pl.ds(n0, NCOL)],
        xv.at[:, pl.ds(halff, halff)],
        sem_b,
    )

    def onehot_planes(c0, c1):
        def body(i, _):
            b = i >> 3
            sl = pl.ds((i & 7) * LANES, LANES)
            ids = xv[b, 0, sl]
            for c in range(c0, c1):
                ov[c, b, sl] = jnp.where(ids == cls[c], one, zero)
            return ()

        plsc.parallel_loop(0, batch * nt, 1, unroll=4, carry=())(body)

    def prop_planes(j0, j1):
        def body(i, _):
            j = j0 + (i >> 7)
            b = (i >> 3) & (batch - 1)
            sl = pl.ds((i & 7) * LANES, LANES)
            ov[NUM_CLASSES + j, b, sl] = xv[b, 1 + j, sl]
            return ()

        plsc.parallel_loop(0, (j1 - j0) * batch * nt, 1, unroll=8, carry=())(
            body
        )

    def out_group(p0, p1):
        return pltpu.async_copy(
            ov.at[pl.ds(p0, p1 - p0)],
            ot_hbm.at[pl.ds(p0, p1 - p0), :, pl.ds(n0, NCOL)],
            sem_o,
        )

    cin0.wait()
    onehot_planes(0, 4)
    g0 = out_group(0, 4)
    onehot_planes(4, 12)
    g1 = out_group(4, 12)
    prop_planes(0, 7)
    g2 = out_group(12, 19)
    cin1.wait()
    prop_planes(7, 11)
    g3 = out_group(19, 23)
    prop_planes(11, 15)
    g4 = out_group(23, 27)
    for g in (g0, g1, g2, g3, g4):
        g.wait()


def kernel(X):
    B, N, F = X.shape
    assert F == NFEAT
    assert N % (NUM_WORKERS * NCOL) == 0 or (B * N) % (NUM_WORKERS * NCOL) == 0

    xt = jnp.transpose(X, (0, 2, 1))
    mesh = plsc.VectorSubcoreMesh(core_axis_name="c", subcore_axis_name="s")
    ot = pl.kernel(
        functools.partial(_sc_body, batch=B),
        out_type=jax.ShapeDtypeStruct((NOUT, B, N), jnp.float32),
        mesh=mesh,
        compiler_params=pltpu.CompilerParams(
            needs_layout_passes=False,
            use_tc_tiling_on_sc=True,
            skip_device_barrier=True,
        ),
        scratch_types=[
            pltpu.VMEM((B, NFEAT, NCOL), jnp.float32),
            pltpu.VMEM((NOUT, B, NCOL), jnp.float32),
            pltpu.SemaphoreType.DMA,
            pltpu.SemaphoreType.DMA,
            pltpu.SemaphoreType.DMA,
        ],
    )(xt)
    return jnp.transpose(ot, (1, 2, 0))
